# baseline (speedup 1.0000x reference)
import functools

import jax
import jax.numpy as jnp
from jax import lax
from jax.experimental import pallas as pl
from jax.experimental.pallas import tpu as pltpu
from jax.experimental.pallas import tpu_sc as plsc

BATCH = 16384
DIM = 64
NC = 2
NS = 16
NW = NC * NS
BPW = BATCH // NW
CHUNK = 128
NCHUNK = BPW // CHUNK


def _sc_gather(idx_h, idx_r, idx_t, entity, translation, norm):
    mesh = plsc.VectorSubcoreMesh(core_axis_name="c", subcore_axis_name="s")
    row_t = jax.ShapeDtypeStruct((BATCH, DIM), jnp.float32)

    @functools.partial(
        pl.kernel,
        mesh=mesh,
        out_type=[row_t, row_t, row_t, row_t],
        compiler_params=pltpu.CompilerParams(use_tc_tiling_on_sc=False),
        scratch_types=[
            pltpu.VMEM((NCHUNK, CHUNK), jnp.int32),
            pltpu.VMEM((NCHUNK, CHUNK), jnp.int32),
            pltpu.VMEM((NCHUNK, CHUNK), jnp.int32),
            pltpu.VMEM((BPW, DIM), jnp.float32),
            pltpu.SemaphoreType.DMA,
        ],
    )
    def k(ih_hbm, ir_hbm, it_hbm, ent_hbm, tr_hbm, nv_hbm,
          oh_hbm, ot_hbm, od_hbm, ow_hbm,
          ih_v, ir_v, it_v, rows_v, sem):
        wid = lax.axis_index("s") * NC + lax.axis_index("c")
        base = wid * BPW
        crow = wid * NCHUNK
        pltpu.sync_copy(ih_hbm.at[pl.ds(crow, NCHUNK)], ih_v)
        pltpu.sync_copy(ir_hbm.at[pl.ds(crow, NCHUNK)], ir_v)
        pltpu.sync_copy(it_hbm.at[pl.ds(crow, NCHUNK)], it_v)

        def one(table_hbm, idx_v, out_hbm):
            handles = []
            for c in range(NCHUNK):
                handles.append(
                    pltpu.async_copy(
                        table_hbm.at[idx_v.at[c]],
                        rows_v.at[pl.ds(c * CHUNK, CHUNK)],
                        sem,
                    )
                )
            for h in handles:
                h.wait()
            pltpu.sync_copy(rows_v, out_hbm.at[pl.ds(base, BPW)])

        one(ent_hbm, ih_v, oh_hbm)
        one(ent_hbm, it_v, ot_hbm)
        one(tr_hbm, ir_v, od_hbm)
        one(nv_hbm, ir_v, ow_hbm)

    return k(idx_h, idx_r, idx_t, entity, translation, norm)


def _score_body(h_ref, t_ref, dr_ref, wr_ref, o_ref):
    h = h_ref[...]
    t = t_ref[...]
    dr = dr_ref[...]
    w = wr_ref[...]
    nw = jnp.sqrt(jnp.sum(w * w, axis=-1, keepdims=True))
    wn = w / jnp.maximum(nw, 1e-12)
    hv = h - jnp.sum(h * wn, axis=-1, keepdims=True) * wn
    tv = t - jnp.sum(t * wn, axis=-1, keepdims=True) * wn
    hn = jnp.sqrt(jnp.sum(hv * hv, axis=-1, keepdims=True))
    hv = hv / jnp.maximum(hn, 1e-12)
    tn = jnp.sqrt(jnp.sum(tv * tv, axis=-1, keepdims=True))
    tv = tv / jnp.maximum(tn, 1e-12)
    diff = hv + dr - tv
    o_ref[...] = jnp.sqrt(jnp.sum(diff * diff, axis=-1))


def _tc_score(h, t, dr, wr):
    bt = 2048
    grid = (BATCH // bt,)
    spec = pl.BlockSpec((bt, DIM), lambda i: (i, 0))
    return pl.pallas_call(
        _score_body,
        grid=grid,
        in_specs=[spec, spec, spec, spec],
        out_specs=pl.BlockSpec((bt,), lambda i: (i,)),
        out_shape=jax.ShapeDtypeStruct((BATCH,), jnp.float32),
    )(h, t, dr, wr)


def kernel(sample, entity_embedding, translation_embedding, norm_vector):
    sample = sample.astype(jnp.int32)
    idx_h = sample[:, 0].reshape(NW * NCHUNK, CHUNK)
    idx_r = sample[:, 1].reshape(NW * NCHUNK, CHUNK)
    idx_t = sample[:, 2].reshape(NW * NCHUNK, CHUNK)
    h, t, dr, wr = _sc_gather(
        idx_h, idx_r, idx_t,
        entity_embedding, translation_embedding, norm_vector,
    )
    return _tc_score(h, t, dr, wr)


# speedup vs baseline: 1.1175x; 1.1175x over previous
import functools

import jax
import jax.numpy as jnp
from jax import lax
from jax.experimental import pallas as pl
from jax.experimental.pallas import tpu as pltpu
from jax.experimental.pallas import tpu_sc as plsc

BATCH = 16384
DIM = 64
NC = 2
NS = 16
NW = NC * NS
BPW = BATCH // NW
CHUNK = 128
NCHUNK = BPW // CHUNK


def _sc_gather(idx_h, idx_r, idx_t, entity, translation, norm):
    mesh = plsc.VectorSubcoreMesh(core_axis_name="c", subcore_axis_name="s")
    row_t = jax.ShapeDtypeStruct((BATCH, 2 * DIM), jnp.float32)

    @functools.partial(
        pl.kernel,
        mesh=mesh,
        out_type=[row_t, row_t, row_t, row_t],
        compiler_params=pltpu.CompilerParams(use_tc_tiling_on_sc=False),
        scratch_types=[
            pltpu.VMEM((NCHUNK, CHUNK), jnp.int32),
            pltpu.VMEM((NCHUNK, CHUNK), jnp.int32),
            pltpu.VMEM((NCHUNK, CHUNK), jnp.int32),
            pltpu.VMEM((BPW, DIM), jnp.float32),
            pltpu.VMEM((BPW, DIM), jnp.float32),
            pltpu.SemaphoreType.DMA,
            pltpu.SemaphoreType.DMA,
            pltpu.SemaphoreType.DMA,
        ],
    )
    def k(ih_hbm, ir_hbm, it_hbm, ent_hbm, tr_hbm, nv_hbm,
          oh_hbm, ot_hbm, od_hbm, ow_hbm,
          ih_v, ir_v, it_v, rows0_v, rows1_v, gsem, ssem0, ssem1):
        wid = lax.axis_index("s") * NC + lax.axis_index("c")
        base = wid * BPW
        crow = wid * NCHUNK
        pltpu.sync_copy(ih_hbm.at[pl.ds(crow, NCHUNK)], ih_v)
        pltpu.sync_copy(ir_hbm.at[pl.ds(crow, NCHUNK)], ir_v)
        pltpu.sync_copy(it_hbm.at[pl.ds(crow, NCHUNK)], it_v)

        rows = (rows0_v, rows1_v)
        ssems = (ssem0, ssem1)
        stages = (
            (ent_hbm, ih_v, oh_hbm),
            (ent_hbm, it_v, ot_hbm),
            (tr_hbm, ir_v, od_hbm),
            (nv_hbm, ir_v, ow_hbm),
        )

        def fire_gather(s):
            table_hbm, idx_v, _ = stages[s]
            buf = rows[s % 2]
            return [
                pltpu.async_copy(
                    table_hbm.at[idx_v.at[c]],
                    buf.at[pl.ds(c * CHUNK, CHUNK)],
                    gsem,
                )
                for c in range(NCHUNK)
            ]

        def fire_store(s):
            out_hbm = stages[s][2]
            buf = rows[s % 2]
            return pltpu.async_copy(
                buf,
                out_hbm.at[pl.ds(base, BPW), pl.ds(0, DIM)],
                ssems[s % 2],
            )

        g0 = fire_gather(0)
        for hnd in g0:
            hnd.wait()
        st0 = fire_store(0)
        g1 = fire_gather(1)
        for hnd in g1:
            hnd.wait()
        st1 = fire_store(1)
        st0.wait()
        g2 = fire_gather(2)
        for hnd in g2:
            hnd.wait()
        st2 = fire_store(2)
        st1.wait()
        g3 = fire_gather(3)
        for hnd in g3:
            hnd.wait()
        st3 = fire_store(3)
        st2.wait()
        st3.wait()

    return k(idx_h, idx_r, idx_t, entity, translation, norm)


def _score_body(h_ref, t_ref, dr_ref, wr_ref, o_ref):
    h = h_ref[...][:, :DIM]
    t = t_ref[...][:, :DIM]
    dr = dr_ref[...][:, :DIM]
    w = wr_ref[...][:, :DIM]
    nw = jnp.sqrt(jnp.sum(w * w, axis=-1, keepdims=True))
    wn = w / jnp.maximum(nw, 1e-12)
    hv = h - jnp.sum(h * wn, axis=-1, keepdims=True) * wn
    tv = t - jnp.sum(t * wn, axis=-1, keepdims=True) * wn
    hn = jnp.sqrt(jnp.sum(hv * hv, axis=-1, keepdims=True))
    hv = hv / jnp.maximum(hn, 1e-12)
    tn = jnp.sqrt(jnp.sum(tv * tv, axis=-1, keepdims=True))
    tv = tv / jnp.maximum(tn, 1e-12)
    diff = hv + dr - tv
    o_ref[...] = jnp.sqrt(jnp.sum(diff * diff, axis=-1))


def _tc_score(h, t, dr, wr):
    bt = 2048
    grid = (BATCH // bt,)
    spec = pl.BlockSpec((bt, 2 * DIM), lambda i: (i, 0))
    return pl.pallas_call(
        _score_body,
        grid=grid,
        in_specs=[spec, spec, spec, spec],
        out_specs=pl.BlockSpec((bt,), lambda i: (i,)),
        out_shape=jax.ShapeDtypeStruct((BATCH,), jnp.float32),
    )(h, t, dr, wr)


def kernel(sample, entity_embedding, translation_embedding, norm_vector):
    sample = sample.astype(jnp.int32)
    idx_h = sample[:, 0].reshape(NW * NCHUNK, CHUNK)
    idx_r = sample[:, 1].reshape(NW * NCHUNK, CHUNK)
    idx_t = sample[:, 2].reshape(NW * NCHUNK, CHUNK)
    h, t, dr, wr = _sc_gather(
        idx_h, idx_r, idx_t,
        entity_embedding, translation_embedding, norm_vector,
    )
    return _tc_score(h, t, dr, wr)
